# Initial kernel scaffold; baseline (speedup 1.0000x reference)
import jax
import jax.numpy as jnp
from jax.experimental import pallas as pl


def kernel(hidden_states, cos, sin, Wq, Wk, Wv, Wo, q_norm_w, k_norm_w):
    raise NotImplementedError("write your pallas kernel here")


# speedup vs baseline: 1.0926x; 1.0926x over previous
import jax
import jax.numpy as jnp
from jax.experimental import pallas as pl

S = 2048
HIDDEN = 2048
NH = 16
NKV = 8
HD = 128
EPS = 1e-6
SCALE = HD ** -0.5
NPROJ = NH + 2 * NKV

BQ = 512
BO = 512
NEG = -1e30


def _qkv_kernel(h_ref, w_ref, nw_ref, cos_ref, sin_ref, out_ref):
    h = pl.program_id(0)
    x = jax.lax.dot_general(
        h_ref[:], w_ref[:], (((1,), (0,)), ((), ())),
        preferred_element_type=jnp.float32)
    var = jnp.mean(jnp.square(x), axis=-1, keepdims=True)
    xn = x * jax.lax.rsqrt(var + EPS) * nw_ref[0, 0]
    x1 = xn[:, : HD // 2]
    x2 = xn[:, HD // 2:]
    rot = jnp.concatenate([-x2, x1], axis=-1)
    roped = xn * cos_ref[:] + rot * sin_ref[:]
    out_ref[0] = jnp.where(h < NH + NKV, roped, x)


def _flash_kernel(q_ref, k_ref, v_ref, out_ref):
    i = pl.program_id(1)
    q = q_ref[0] * SCALE
    acc0 = jnp.zeros((BQ, HD), jnp.float32)
    m0 = jnp.full((BQ, 1), NEG, jnp.float32)
    l0 = jnp.zeros((BQ, 1), jnp.float32)

    def body(j, carry):
        acc, m, l = carry
        kj = k_ref[0, pl.ds(j * BQ, BQ), :]
        vj = v_ref[0, pl.ds(j * BQ, BQ), :]
        s = jax.lax.dot_general(
            q, kj, (((1,), (1,)), ((), ())),
            preferred_element_type=jnp.float32)
        row = i * BQ + jax.lax.broadcasted_iota(jnp.int32, (BQ, BQ), 0)
        col = j * BQ + jax.lax.broadcasted_iota(jnp.int32, (BQ, BQ), 1)
        s = jnp.where(col <= row, s, NEG)
        m_new = jnp.maximum(m, jnp.max(s, axis=-1, keepdims=True))
        alpha = jnp.exp(m - m_new)
        p = jnp.exp(s - m_new)
        l_new = l * alpha + jnp.sum(p, axis=-1, keepdims=True)
        pv = jax.lax.dot_general(
            p, vj, (((1,), (0,)), ((), ())),
            preferred_element_type=jnp.float32)
        return acc * alpha + pv, m_new, l_new

    acc, _, l = jax.lax.fori_loop(0, i + 1, body, (acc0, m0, l0))
    out_ref[0] = acc / l


def _oproj_kernel(a_ref, w_ref, out_ref):
    @pl.when(pl.program_id(1) == 0)
    def _():
        out_ref[:] = jnp.zeros_like(out_ref)

    out_ref[:] += jax.lax.dot_general(
        a_ref[0], w_ref[0], (((1,), (0,)), ((), ())),
        preferred_element_type=jnp.float32)


def kernel(hidden_states, cos, sin, Wq, Wk, Wv, Wo, q_norm_w, k_norm_w):
    hs = hidden_states[0]
    cos0 = cos[0]
    sin0 = sin[0]

    w_all = jnp.concatenate([Wq, Wk, Wv], axis=1)
    nw = jnp.concatenate([
        jnp.broadcast_to(q_norm_w, (NH, HD)),
        jnp.broadcast_to(k_norm_w, (NKV, HD)),
        jnp.ones((NKV, HD), jnp.float32),
    ], axis=0).reshape(NPROJ, 1, HD)

    qkv = pl.pallas_call(
        _qkv_kernel,
        grid=(NPROJ,),
        in_specs=[
            pl.BlockSpec((S, HIDDEN), lambda h: (0, 0)),
            pl.BlockSpec((HIDDEN, HD), lambda h: (0, h)),
            pl.BlockSpec((1, 1, HD), lambda h: (h, 0, 0)),
            pl.BlockSpec((S, HD), lambda h: (0, 0)),
            pl.BlockSpec((S, HD), lambda h: (0, 0)),
        ],
        out_specs=pl.BlockSpec((1, S, HD), lambda h: (h, 0, 0)),
        out_shape=jax.ShapeDtypeStruct((NPROJ, S, HD), jnp.float32),
    )(hs, w_all, nw, cos0, sin0)

    attn = pl.pallas_call(
        _flash_kernel,
        grid=(NH, S // BQ),
        in_specs=[
            pl.BlockSpec((1, BQ, HD), lambda h, i: (h, i, 0)),
            pl.BlockSpec((1, S, HD), lambda h, i: (NH + h // 2, 0, 0)),
            pl.BlockSpec((1, S, HD), lambda h, i: (NH + NKV + h // 2, 0, 0)),
        ],
        out_specs=pl.BlockSpec((1, BQ, HD), lambda h, i: (h, i, 0)),
        out_shape=jax.ShapeDtypeStruct((NH, S, HD), jnp.float32),
    )(qkv, qkv, qkv)

    wo = Wo.reshape(NH, HD, HIDDEN)
    out = pl.pallas_call(
        _oproj_kernel,
        grid=(S // BO, NH),
        in_specs=[
            pl.BlockSpec((1, BO, HD), lambda i, h: (h, i, 0)),
            pl.BlockSpec((1, HD, HIDDEN), lambda i, h: (h, 0, 0)),
        ],
        out_specs=pl.BlockSpec((BO, HIDDEN), lambda i, h: (i, 0)),
        out_shape=jax.ShapeDtypeStruct((S, HIDDEN), jnp.float32),
    )(attn, wo)

    return out[None]
